# baseline (speedup 1.0000x reference)
import functools

import jax
import jax.numpy as jnp
from jax import lax
from jax.experimental import pallas as pl
from jax.experimental.pallas import tpu as pltpu
from jax.experimental.pallas import tpu_sc as plsc

B = 16384
D = 32
BLK = 8
NROW = 1000000
NC = 2
NS = 16
L = 16
NW = NC * NS
BPW = B // NW
NG = BPW // L

_mesh = plsc.VectorSubcoreMesh(
    core_axis_name="c", subcore_axis_name="s", num_cores=NC, num_subcores=NS
)


@functools.partial(
    pl.kernel,
    out_type=(
        jax.ShapeDtypeStruct((B,), jnp.float32),
        jax.ShapeDtypeStruct((B,), jnp.float32),
    ),
    mesh=_mesh,
    compiler_params=pltpu.CompilerParams(needs_layout_passes=False),
    scratch_types=(
        pltpu.VMEM((BPW,), jnp.int32),
        pltpu.VMEM((BPW,), jnp.int32),
        pltpu.VMEM((BPW,), jnp.int32),
        pltpu.VMEM((L, BLK, D), jnp.float32),
        pltpu.VMEM((L, BLK, D), jnp.float32),
        pltpu.VMEM((L, BLK, D), jnp.float32),
        pltpu.VMEM((BPW,), jnp.float32),
        pltpu.VMEM((BPW,), jnp.float32),
        pltpu.SemaphoreType.DMA,
    ),
)
def _bpr_scores(uix, pix, nix, utab, mtab, pos_out, neg_out,
                idx_u, idx_p, idx_n, blocks_u, blocks_p, blocks_n,
                sc_pos, sc_neg, sem):
    wid = lax.axis_index("s") * NC + lax.axis_index("c")
    base = wid * BPW
    utiles = utab.reshape(NROW // BLK, BLK, D)
    mtiles = mtab.reshape(NROW // BLK, BLK, D)

    pltpu.sync_copy(uix.at[pl.ds(base, BPW)], idx_u)
    pltpu.sync_copy(pix.at[pl.ds(base, BPW)], idx_p)
    pltpu.sync_copy(nix.at[pl.ds(base, BPW)], idx_n)

    iot = lax.iota(jnp.int32, L)
    onehot = [iot == rr for rr in range(L)]

    def group_body(g, carry):
        su_v = idx_u[pl.ds(g * L, L)]
        sp_v = idx_p[pl.ds(g * L, L)]
        sn_v = idx_n[pl.ds(g * L, L)]
        bu_v = jax.lax.shift_right_logical(su_v, 3)
        bp_v = jax.lax.shift_right_logical(sp_v, 3)
        bn_v = jax.lax.shift_right_logical(sn_v, 3)
        ru_v = su_v & (BLK - 1)
        rp_v = sp_v & (BLK - 1)
        rn_v = sn_v & (BLK - 1)
        cps = []
        for rr in range(L):
            cps.append(pltpu.async_copy(
                utiles.at[bu_v[rr]], blocks_u.at[rr], sem))
            cps.append(pltpu.async_copy(
                mtiles.at[bp_v[rr]], blocks_p.at[rr], sem))
            cps.append(pltpu.async_copy(
                mtiles.at[bn_v[rr]], blocks_n.at[rr], sem))
        for cp in cps:
            cp.wait()

        accp = jnp.zeros((L,), jnp.float32)
        accn = jnp.zeros((L,), jnp.float32)
        for rr in range(L):
            u0 = blocks_u[rr, ru_v[rr], pl.ds(0, L)]
            u1 = blocks_u[rr, ru_v[rr], pl.ds(L, L)]
            p0 = blocks_p[rr, rp_v[rr], pl.ds(0, L)]
            p1 = blocks_p[rr, rp_v[rr], pl.ds(L, L)]
            n0 = blocks_n[rr, rn_v[rr], pl.ds(0, L)]
            n1 = blocks_n[rr, rn_v[rr], pl.ds(L, L)]
            sp = jnp.sum(u0 * p0 + u1 * p1)
            sn = jnp.sum(u0 * n0 + u1 * n1)
            accp = jnp.where(onehot[rr], sp, accp)
            accn = jnp.where(onehot[rr], sn, accn)
        sc_pos[pl.ds(g * L, L)] = accp
        sc_neg[pl.ds(g * L, L)] = accn
        return carry

    lax.fori_loop(0, NG, group_body, 0)

    pltpu.sync_copy(sc_pos, pos_out.at[pl.ds(base, BPW)])
    pltpu.sync_copy(sc_neg, neg_out.at[pl.ds(base, BPW)])


def kernel(user_idxs, pos_idxs, neg_idxs, user_table, movie_table):
    return _bpr_scores(user_idxs, pos_idxs, neg_idxs, user_table, movie_table)


# speedup vs baseline: 1.0962x; 1.0962x over previous
import functools

import jax
import jax.numpy as jnp
from jax import lax
from jax.experimental import pallas as pl
from jax.experimental.pallas import tpu as pltpu
from jax.experimental.pallas import tpu_sc as plsc

B = 16384
D = 32
NC = 2
NS = 16
L = 16
NW = NC * NS
BPW = B // NW
NG = BPW // L

_mesh = plsc.VectorSubcoreMesh(
    core_axis_name="c", subcore_axis_name="s", num_cores=NC, num_subcores=NS
)


@functools.partial(
    pl.kernel,
    out_type=(
        jax.ShapeDtypeStruct((B,), jnp.float32),
        jax.ShapeDtypeStruct((B,), jnp.float32),
    ),
    mesh=_mesh,
    compiler_params=pltpu.CompilerParams(needs_layout_passes=False),
    scratch_types=(
        pltpu.VMEM((BPW,), jnp.int32),
        pltpu.VMEM((BPW,), jnp.int32),
        pltpu.VMEM((BPW,), jnp.int32),
        pltpu.VMEM((L, D), jnp.float32),
        pltpu.VMEM((L, D), jnp.float32),
        pltpu.VMEM((L, D), jnp.float32),
        pltpu.VMEM((BPW,), jnp.float32),
        pltpu.VMEM((BPW,), jnp.float32),
        pltpu.SemaphoreType.DMA,
        pltpu.SemaphoreType.DMA,
        pltpu.SemaphoreType.DMA,
        pltpu.SemaphoreType.DMA,
    ),
)
def _bpr_scores(uix, pix, nix, utab, mtab, pos_out, neg_out,
                idx_u, idx_p, idx_n, rows_u, rows_p, rows_n,
                sc_pos, sc_neg, sem0, sem1, sem2, sem3):
    sems = (sem0, sem1, sem2, sem3)
    wid = lax.axis_index("s") * NC + lax.axis_index("c")
    base = wid * BPW

    pltpu.sync_copy(uix.at[pl.ds(base, BPW)], idx_u)
    pltpu.sync_copy(pix.at[pl.ds(base, BPW)], idx_p)
    pltpu.sync_copy(nix.at[pl.ds(base, BPW)], idx_n)

    iot = lax.iota(jnp.int32, L)
    onehot = [iot == rr for rr in range(L)]

    def group_body(g, carry):
        su_v = idx_u[pl.ds(g * L, L)]
        sp_v = idx_p[pl.ds(g * L, L)]
        sn_v = idx_n[pl.ds(g * L, L)]
        cps = []
        for rr in range(L):
            cps.append(pltpu.async_copy(utab.at[su_v[rr]], rows_u.at[rr], sems[rr % 4]))
            cps.append(pltpu.async_copy(mtab.at[sp_v[rr]], rows_p.at[rr], sems[(rr + 1) % 4]))
            cps.append(pltpu.async_copy(mtab.at[sn_v[rr]], rows_n.at[rr], sems[(rr + 2) % 4]))
        for cp in cps:
            cp.wait()

        accp = jnp.zeros((L,), jnp.float32)
        accn = jnp.zeros((L,), jnp.float32)
        for rr in range(L):
            u0 = rows_u[rr, pl.ds(0, L)]
            u1 = rows_u[rr, pl.ds(L, L)]
            p0 = rows_p[rr, pl.ds(0, L)]
            p1 = rows_p[rr, pl.ds(L, L)]
            n0 = rows_n[rr, pl.ds(0, L)]
            n1 = rows_n[rr, pl.ds(L, L)]
            sp = jnp.sum(u0 * p0 + u1 * p1)
            sn = jnp.sum(u0 * n0 + u1 * n1)
            accp = jnp.where(onehot[rr], sp, accp)
            accn = jnp.where(onehot[rr], sn, accn)
        sc_pos[pl.ds(g * L, L)] = accp
        sc_neg[pl.ds(g * L, L)] = accn
        return carry

    lax.fori_loop(0, NG, group_body, 0)

    pltpu.sync_copy(sc_pos, pos_out.at[pl.ds(base, BPW)])
    pltpu.sync_copy(sc_neg, neg_out.at[pl.ds(base, BPW)])


def kernel(user_idxs, pos_idxs, neg_idxs, user_table, movie_table):
    return _bpr_scores(user_idxs, pos_idxs, neg_idxs, user_table, movie_table)
